# Initial kernel scaffold; baseline (speedup 1.0000x reference)
import jax
import jax.numpy as jnp
from jax.experimental import pallas as pl


def kernel(idsTensor, table, W, b):
    raise NotImplementedError("write your pallas kernel here")


# speedup vs baseline: 1.4111x; 1.4111x over previous
import functools

import jax
import jax.numpy as jnp
from jax import lax
from jax.experimental import pallas as pl
from jax.experimental.pallas import tpu as pltpu
from jax.experimental.pallas import tpu_sc as plsc

_EMB = 32
_B = 16384
_NC = 2
_NS = 16
_NW = _NC * _NS
_BPW = _B // _NW
_NBLK = _BPW // 16
_CHUNK = 128
_NCHUNK = _BPW // _CHUNK
_SLOT = 32


def _sc_body(idx1_hbm, idx2_hbm, sd_hbm, table_hbm, wb_hbm, out_hbm,
             idx1_v, idx2_v, sd_v, rows1_v, rows2_v, w_v, red_v, o_v, sem):
    wid = lax.axis_index("s") * _NC + lax.axis_index("c")
    base = wid * _BPW
    lane = lax.iota(jnp.int32, 16)
    zeros = jnp.zeros((16,), jnp.float32)

    pltpu.sync_copy(idx1_hbm.at[pl.ds(base, _BPW)], idx1_v)
    pltpu.sync_copy(idx2_hbm.at[pl.ds(base, _BPW)], idx2_v)
    pltpu.sync_copy(sd_hbm.at[pl.ds(base, _BPW)], sd_v)
    pltpu.sync_copy(wb_hbm, w_v)

    copies = []
    for j in range(_NCHUNK):
        rsl = pl.ds(j * _CHUNK, _CHUNK)
        copies.append(pltpu.async_copy(
            table_hbm.at[idx1_v.at[rsl]], rows1_v.at[rsl], sem))
        copies.append(pltpu.async_copy(
            table_hbm.at[idx2_v.at[rsl]], rows2_v.at[rsl], sem))

    for j in range(16 * _SLOT // 16):
        red_v[pl.ds(16 * j, 16)] = zeros

    w1lo = w_v[pl.ds(0, 16)]
    w1hi = w_v[pl.ds(16, 16)]
    w2lo = w_v[pl.ds(32, 16)]
    w2hi = w_v[pl.ds(48, 16)]
    wtail = w_v[pl.ds(64, 16)]
    w_sd = wtail[0]
    b0 = wtail[1]
    masks = [lane == r for r in range(16)]

    for cp in copies:
        cp.wait()

    def block(i, _):
        sl = pl.ds(i * 16, 16)
        acc = sd_v[sl] * w_sd + b0
        for rr in range(16):
            r = i * 16 + rr
            v = (rows1_v[r, pl.ds(0, 16)] * w1lo
                 + rows1_v[r, pl.ds(16, 16)] * w1hi
                 + rows2_v[r, pl.ds(0, 16)] * w2lo
                 + rows2_v[r, pl.ds(16, 16)] * w2hi)
            s = rr * _SLOT
            red_v[pl.ds(s, 16)] = v
            v = v + red_v[pl.ds(s + 8, 16)]
            red_v[pl.ds(s, 16)] = v
            v = v + red_v[pl.ds(s + 4, 16)]
            red_v[pl.ds(s, 16)] = v
            v = v + red_v[pl.ds(s + 2, 16)]
            red_v[pl.ds(s, 16)] = v
            v = v + red_v[pl.ds(s + 1, 16)]
            acc = acc + jnp.where(masks[rr], v[0], 0.0)
        o_v[sl] = 1.0 / (1.0 + jnp.exp(-acc))
        return _

    lax.fori_loop(0, _NBLK, block, None)

    pltpu.sync_copy(o_v, out_hbm.at[pl.ds(base, _BPW)])


@jax.jit
def _sc_forward(idx1, idx2, sd, table, wb):
    mesh = plsc.VectorSubcoreMesh(core_axis_name="c", subcore_axis_name="s")
    f = functools.partial(
        pl.kernel,
        mesh=mesh,
        compiler_params=pltpu.CompilerParams(use_tc_tiling_on_sc=False),
        out_type=jax.ShapeDtypeStruct((_B,), jnp.float32),
        scratch_types=[
            pltpu.VMEM((_BPW,), jnp.int32),
            pltpu.VMEM((_BPW,), jnp.int32),
            pltpu.VMEM((_BPW,), jnp.float32),
            pltpu.VMEM((_BPW, _EMB), jnp.float32),
            pltpu.VMEM((_BPW, _EMB), jnp.float32),
            pltpu.VMEM((80,), jnp.float32),
            pltpu.VMEM((16 * _SLOT,), jnp.float32),
            pltpu.VMEM((_BPW,), jnp.float32),
            pltpu.SemaphoreType.DMA,
        ],
    )(_sc_body)
    return f(idx1, idx2, sd, table, wb)


def kernel(idsTensor, table, W, b):
    idx1 = idsTensor[:, 0].astype(jnp.int32)
    idx2 = idsTensor[:, 1].astype(jnp.int32)
    sd = idsTensor[:, 2]
    wb = jnp.concatenate(
        [W.reshape(-1), b, jnp.zeros((14,), jnp.float32)])
    out = _sc_forward(idx1, idx2, sd, table, wb)
    return out.reshape(_B, 1)
